# baseline (speedup 1.0000x reference)
import functools

import jax
import jax.numpy as jnp
from jax import lax
from jax.experimental import pallas as pl
from jax.experimental.pallas import tpu as pltpu
from jax.experimental.pallas import tpu_sc as plsc

N_TOK = 16384
K_DIM = 50
DIM = 32
SCALE = 10.0

NC = 2
NS = 16
NW = NC * NS

CHUNK = 128
N_UNITS = N_TOK * K_DIM // CHUNK
U_PER_W = N_UNITS // NW
NB = N_TOK // CHUNK


@jax.jit
def _scaled_embedding(idx_lin, table):
    mesh = plsc.VectorSubcoreMesh(core_axis_name="c", subcore_axis_name="s")

    @functools.partial(
        pl.kernel,
        mesh=mesh,
        out_type=jax.ShapeDtypeStruct((K_DIM, 4, NB, 8, CHUNK), jnp.float32),
        scratch_types=[
            pltpu.VMEM((U_PER_W, CHUNK), jnp.int32),
            pltpu.VMEM((2, CHUNK, DIM), jnp.float32),
            pltpu.VMEM((2, 4, 8, CHUNK), jnp.float32),
            pltpu.SemaphoreType.DMA,
            pltpu.SemaphoreType.DMA,
            pltpu.SemaphoreType.DMA,
            pltpu.SemaphoreType.DMA,
        ],
        compiler_params=pltpu.CompilerParams(
            use_tc_tiling_on_sc=False, needs_layout_passes=False
        ),
    )
    def k(idx_hbm, table_hbm, out_hbm, idx_v, gbuf, tbuf, sg0, sg1, ss0, ss1):
        wid = lax.axis_index("s") * NC + lax.axis_index("c")
        sem_g = (sg0, sg1)
        sem_s = (ss0, ss1)
        u0 = wid * U_PER_W
        pltpu.sync_copy(idx_hbm.at[pl.ds(u0, U_PER_W)], idx_v)

        def fire_gather(j, b):
            pltpu.async_copy(table_hbm.at[idx_v.at[j]], gbuf.at[b], sem_g[b])

        def drain_gather(b):
            pltpu.make_async_copy(
                table_hbm.at[idx_v.at[0]], gbuf.at[b], sem_g[b]
            ).wait()

        def fire_out(j, b):
            u = u0 + j
            kk = lax.shift_right_logical(u, 7)
            nb = lax.bitwise_and(u, NB - 1)
            for tr in range(4):
                pltpu.async_copy(
                    tbuf.at[b, tr], out_hbm.at[kk, tr, nb], sem_s[b]
                )

        def drain_out(b):
            for tr in range(4):
                pltpu.make_async_copy(
                    tbuf.at[b, tr], out_hbm.at[0, tr, 0], sem_s[b]
                ).wait()

        iota = lax.iota(jnp.int32, 16)

        def transpose_scale(b):
            def dloop(d, carry):
                dsp = jnp.full((16,), d, jnp.int32)
                tr = lax.shift_right_logical(d, 3)
                a = lax.bitwise_and(d, 7)
                for g in range(8):
                    nv = iota + (g * 16)
                    vals = plsc.load_gather(gbuf.at[b], [nv, dsp])
                    tbuf[b, tr, a, pl.ds(g * 16, 16)] = vals * SCALE
                return carry

            lax.fori_loop(0, DIM, dloop, 0, unroll=4)

        def step(j, b):
            drain_gather(b)

            @pl.when(jnp.logical_and(j >= 1, j < U_PER_W - 1))
            def _():
                drain_out(1 - b)

            @pl.when(j < U_PER_W - 1)
            def _():
                fire_gather(j + 1, 1 - b)

            transpose_scale(b)
            fire_out(j, b)

        fire_gather(0, 0)

        def pair(i, carry):
            step(2 * i, 0)
            step(2 * i + 1, 1)
            return carry

        lax.fori_loop(0, U_PER_W // 2, pair, 0)
        drain_out(0)
        drain_out(1)

    return k(idx_lin, table)


def kernel(inputs, table):
    idx_lin = inputs.T.reshape(N_UNITS, CHUNK).astype(jnp.int32)
    out5 = _scaled_embedding(idx_lin, table)
    return out5.transpose(2, 4, 0, 1, 3).reshape(N_TOK, K_DIM, DIM)


# speedup vs baseline: 1.2763x; 1.2763x over previous
import functools

import jax
import jax.numpy as jnp
from jax import lax
from jax.experimental import pallas as pl
from jax.experimental.pallas import tpu as pltpu
from jax.experimental.pallas import tpu_sc as plsc

N_TOK = 16384
K_DIM = 50
DIM = 32
SCALE = 10.0

NC = 2
NS = 16
NW = NC * NS

CHUNK = 128
N_UNITS = N_TOK * K_DIM // CHUNK
U_PER_W = N_UNITS // NW
NB = N_TOK // CHUNK


@jax.jit
def _scaled_embedding(idx_lin, table):
    mesh = plsc.VectorSubcoreMesh(core_axis_name="c", subcore_axis_name="s")

    @functools.partial(
        pl.kernel,
        mesh=mesh,
        out_type=jax.ShapeDtypeStruct((K_DIM, 4, NB, 8, CHUNK), jnp.float32),
        scratch_types=[
            pltpu.VMEM((U_PER_W, CHUNK), jnp.int32),
            pltpu.VMEM((2, CHUNK, DIM), jnp.float32),
            pltpu.VMEM((2, CHUNK * (DIM + 1)), jnp.float32),
            pltpu.VMEM((2, 4, 8, CHUNK), jnp.float32),
            pltpu.SemaphoreType.DMA,
            pltpu.SemaphoreType.DMA,
            pltpu.SemaphoreType.DMA,
            pltpu.SemaphoreType.DMA,
        ],
        compiler_params=pltpu.CompilerParams(
            use_tc_tiling_on_sc=False, needs_layout_passes=False
        ),
    )
    def k(idx_hbm, table_hbm, out_hbm, idx_v, gbuf, sbuf, tbuf, sg0, sg1, ss0, ss1):
        wid = lax.axis_index("s") * NC + lax.axis_index("c")
        sem_g = (sg0, sg1)
        sem_s = (ss0, ss1)
        u0 = wid * U_PER_W
        pltpu.sync_copy(idx_hbm.at[pl.ds(u0, U_PER_W)], idx_v)

        def fire_gather(j, b):
            pltpu.async_copy(table_hbm.at[idx_v.at[j]], gbuf.at[b], sem_g[b])

        def drain_gather(b):
            pltpu.make_async_copy(
                table_hbm.at[idx_v.at[0]], gbuf.at[b], sem_g[b]
            ).wait()

        def fire_out(j, b):
            u = u0 + j
            kk = lax.shift_right_logical(u, 7)
            nb = lax.bitwise_and(u, NB - 1)
            for tr in range(4):
                pltpu.async_copy(
                    tbuf.at[b, tr], out_hbm.at[kk, tr, nb], sem_s[b]
                )

        def drain_out(b):
            for tr in range(4):
                pltpu.make_async_copy(
                    tbuf.at[b, tr], out_hbm.at[0, tr, 0], sem_s[b]
                ).wait()

        iota = lax.iota(jnp.int32, 16)
        SK = DIM + 1
        skew_base = [(iota + 16 * g) * SK for g in range(8)]

        def transpose_scale(b):
            def rloop(n, carry):
                base = n * SK
                sbuf[b, pl.ds(base, 16)] = gbuf[b, n, 0:16]
                sbuf[b, pl.ds(base + 16, 16)] = gbuf[b, n, 16:32]
                return carry

            lax.fori_loop(0, CHUNK, rloop, 0, unroll=8)

            def dloop(d, carry):
                tr = lax.shift_right_logical(d, 3)
                a = lax.bitwise_and(d, 7)
                for g in range(8):
                    vals = plsc.load_gather(sbuf.at[b], [skew_base[g] + d])
                    tbuf[b, tr, a, pl.ds(g * 16, 16)] = vals * SCALE
                return carry

            lax.fori_loop(0, DIM, dloop, 0, unroll=4)

        def step(j, b):
            drain_gather(b)

            @pl.when(jnp.logical_and(j >= 1, j < U_PER_W - 1))
            def _():
                drain_out(1 - b)

            @pl.when(j < U_PER_W - 1)
            def _():
                fire_gather(j + 1, 1 - b)

            transpose_scale(b)
            fire_out(j, b)

        fire_gather(0, 0)

        def pair(i, carry):
            step(2 * i, 0)
            step(2 * i + 1, 1)
            return carry

        lax.fori_loop(0, U_PER_W // 2, pair, 0)
        drain_out(0)
        drain_out(1)

    return k(idx_lin, table)


def kernel(inputs, table):
    idx_lin = inputs.T.reshape(N_UNITS, CHUNK).astype(jnp.int32)
    out5 = _scaled_embedding(idx_lin, table)
    return out5.transpose(2, 4, 0, 1, 3).reshape(N_TOK, K_DIM, DIM)


# speedup vs baseline: 1.7853x; 1.3988x over previous
import functools

import jax
import jax.numpy as jnp
from jax import lax
from jax.experimental import pallas as pl
from jax.experimental.pallas import tpu as pltpu
from jax.experimental.pallas import tpu_sc as plsc

N_TOK = 16384
K_DIM = 50
DIM = 32
SCALE = 10.0

NC = 2
NS = 16
NW = NC * NS

CHUNK = 128
N_UNITS = N_TOK * K_DIM // CHUNK
U_PER_W = N_UNITS // NW
NB = N_TOK // CHUNK


@jax.jit
def _scaled_embedding(idx_lin, table):
    mesh = plsc.VectorSubcoreMesh(core_axis_name="c", subcore_axis_name="s")

    @functools.partial(
        pl.kernel,
        mesh=mesh,
        out_type=jax.ShapeDtypeStruct((K_DIM, 4, NB, 8, CHUNK), jnp.float32),
        scratch_types=[
            pltpu.VMEM((U_PER_W, CHUNK), jnp.int32),
            pltpu.VMEM((2, CHUNK, DIM), jnp.float32),
            pltpu.VMEM((2, CHUNK * (DIM + 1)), jnp.float32),
            pltpu.VMEM((2, 4, 8, CHUNK), jnp.float32),
            pltpu.SemaphoreType.DMA,
            pltpu.SemaphoreType.DMA,
            pltpu.SemaphoreType.DMA,
            pltpu.SemaphoreType.DMA,
        ],
        compiler_params=pltpu.CompilerParams(
            use_tc_tiling_on_sc=False, needs_layout_passes=False
        ),
    )
    def k(idx_hbm, table_hbm, out_hbm, idx_v, gbuf, sbuf, tbuf, sg0, sg1, ss0, ss1):
        wid = lax.axis_index("s") * NC + lax.axis_index("c")
        sem_g = (sg0, sg1)
        sem_s = (ss0, ss1)
        u0 = wid * U_PER_W
        pltpu.sync_copy(idx_hbm.at[pl.ds(u0, U_PER_W)], idx_v)

        def fire_gather(j, b):
            pltpu.async_copy(table_hbm.at[idx_v.at[j]], gbuf.at[b], sem_g[b])

        def drain_gather(b):
            pltpu.make_async_copy(
                table_hbm.at[idx_v.at[0]], gbuf.at[b], sem_g[b]
            ).wait()

        def fire_out(j, b):
            u = u0 + j
            kk = lax.shift_right_logical(u, 7)
            nb = lax.bitwise_and(u, NB - 1)
            for tr in range(4):
                pltpu.async_copy(
                    tbuf.at[b, tr], out_hbm.at[kk, tr, nb], sem_s[b]
                )

        def drain_out(b):
            for tr in range(4):
                pltpu.make_async_copy(
                    tbuf.at[b, tr], out_hbm.at[0, tr, 0], sem_s[b]
                ).wait()

        iota = lax.iota(jnp.int32, 16)
        SK = DIM + 1
        skew_base = [(iota + 16 * g) * SK for g in range(8)]

        def transpose_scale(b):
            @plsc.parallel_loop(0, CHUNK, unroll=8)
            def rloop(n):
                base = n * SK
                sbuf[b, pl.ds(base, 16)] = gbuf[b, n, 0:16]
                sbuf[b, pl.ds(base + 16, 16)] = gbuf[b, n, 16:32]

            @plsc.parallel_loop(0, DIM, unroll=4)
            def dloop(d):
                tr = lax.shift_right_logical(d, 3)
                a = lax.bitwise_and(d, 7)
                for g in range(8):
                    vals = plsc.load_gather(sbuf.at[b], [skew_base[g] + d])
                    tbuf[b, tr, a, pl.ds(g * 16, 16)] = vals * SCALE

        def step(j, b):
            drain_gather(b)

            @pl.when(jnp.logical_and(j >= 1, j < U_PER_W - 1))
            def _():
                drain_out(1 - b)

            @pl.when(j < U_PER_W - 1)
            def _():
                fire_gather(j + 1, 1 - b)

            transpose_scale(b)
            fire_out(j, b)

        fire_gather(0, 0)

        def pair(i, carry):
            step(2 * i, 0)
            step(2 * i + 1, 1)
            return carry

        lax.fori_loop(0, U_PER_W // 2, pair, 0)
        drain_out(0)
        drain_out(1)

    return k(idx_lin, table)


def kernel(inputs, table):
    idx_lin = inputs.T.reshape(N_UNITS, CHUNK).astype(jnp.int32)
    out5 = _scaled_embedding(idx_lin, table)
    return out5.transpose(2, 4, 0, 1, 3).reshape(N_TOK, K_DIM, DIM)
